# baseline (speedup 1.0000x reference)
import jax
import jax.numpy as jnp
from jax import lax
from jax.experimental import pallas as pl
from jax.experimental.pallas import tpu as pltpu
from jax.experimental.pallas import tpu_sc as plsc

_NC = 2
_NS = 16
_NW = _NC * _NS
_L = 16

_BATCH = 4096
_BPW = _BATCH // _NW
_DIM = 128
_HALF = 64
_NUM_REL = 1000
_CHUNK = 64

_S1 = 0.9999788726879895
_S3 = -0.16649714106979646
_S5 = 0.007992247366759672
_C0 = 0.9999998110259923
_C2 = -0.49999394332144725
_C4 = 0.0416363038739887
_C6 = -0.001340053632153032


def _sc_body(h_hbm, r_hbm, t_hbm, ent_hbm, rel2_hbm, out_hbm,
             h_idx, r_idx, t_idx, r2_idx, h_rows, t_rows, p_rows, out_v,
             sem_h, sem_t, sem_p):
    wid = lax.axis_index("s") * _NC + lax.axis_index("c")
    base = wid * _BPW

    pltpu.sync_copy(h_hbm.at[pl.ds(base, _BPW)], h_idx)
    pltpu.sync_copy(r_hbm.at[pl.ds(base, _BPW)], r_idx)
    pltpu.sync_copy(t_hbm.at[pl.ds(base, _BPW)], t_idx)

    for k in range(_BPW // _L):
        r2_idx[pl.ds(k * _L, _L)] = r_idx[pl.ds(k * _L, _L)] >> 1

    _CH = _BPW // _CHUNK
    dh, dt, dp = [], [], []
    for c in range(_CH):
        sl = pl.ds(c * _CHUNK, _CHUNK)
        dh.append(pltpu.async_copy(ent_hbm.at[h_idx.at[sl]], h_rows.at[sl], sem_h))
        dt.append(pltpu.async_copy(ent_hbm.at[t_idx.at[sl]], t_rows.at[sl], sem_t))
        dp.append(pltpu.async_copy(rel2_hbm.at[r2_idx.at[sl]], p_rows.at[sl], sem_p))

    lanes = lax.iota(jnp.int32, _L)
    gpc = _CHUNK // _L

    for g in range(_BPW // _L):
        if g % gpc == 0:
            c = g // gpc
            dh[c].wait()
            dt[c].wait()
            dp[c].wait()
        rows = g * _L + lanes
        rvals = r_idx[pl.ds(g * _L, _L)]
        p_off = (rvals & 1) << 6

        def d_step(d, acc, rows=rows, p_off=p_off):
            dcol = (lanes + d) & (_HALF - 1)
            h_re = plsc.load_gather(h_rows, [rows, dcol])
            h_im = plsc.load_gather(h_rows, [rows, dcol + _HALF])
            t_re = plsc.load_gather(t_rows, [rows, dcol])
            t_im = plsc.load_gather(t_rows, [rows, dcol + _HALF])
            ph = plsc.load_gather(p_rows, [rows, p_off + dcol])
            x2 = ph * ph
            s = ph * (_S1 + x2 * (_S3 + x2 * _S5))
            c = _C0 + x2 * (_C2 + x2 * (_C4 + x2 * _C6))
            d_re = h_re * c - h_im * s - t_re
            d_im = h_re * s + h_im * c - t_im
            return acc + d_re * d_re + d_im * d_im

        acc = lax.fori_loop(0, _HALF, d_step, jnp.zeros((_L,), jnp.float32),
                            unroll=4)

        bits = plsc.bitcast(acc, jnp.int32)
        y = plsc.bitcast(jnp.int32(0x5F3759DF) - (bits >> 1), jnp.float32)
        for _ in range(3):
            y = y * (1.5 - 0.5 * acc * y * y)
        root = jnp.where(acc > 0.0, acc * y, 0.0)
        out_v[pl.ds(g * _L, _L)] = -root

    pltpu.sync_copy(out_v, out_hbm.at[pl.ds(base, _BPW)])


_sc_kernel = pl.kernel(
    _sc_body,
    out_type=jax.ShapeDtypeStruct((_BATCH,), jnp.float32),
    mesh=plsc.VectorSubcoreMesh(
        core_axis_name="c", subcore_axis_name="s",
        num_cores=_NC, num_subcores=_NS),
    scratch_types=[
        pltpu.VMEM((_BPW,), jnp.int32),
        pltpu.VMEM((_BPW,), jnp.int32),
        pltpu.VMEM((_BPW,), jnp.int32),
        pltpu.VMEM((_BPW,), jnp.int32),
        pltpu.VMEM((_BPW, _DIM), jnp.float32),
        pltpu.VMEM((_BPW, _DIM), jnp.float32),
        pltpu.VMEM((_BPW, _DIM), jnp.float32),
        pltpu.VMEM((_BPW,), jnp.float32),
        pltpu.SemaphoreType.DMA,
        pltpu.SemaphoreType.DMA,
        pltpu.SemaphoreType.DMA,
    ],
    compiler_params=pltpu.CompilerParams(needs_layout_passes=False),
)


@jax.jit
def kernel(h, r, t, entity_embedding, relation_embedding):
    rel2 = relation_embedding.reshape(_NUM_REL // 2, _DIM)
    return _sc_kernel(h.astype(jnp.int32), r.astype(jnp.int32),
                      t.astype(jnp.int32), entity_embedding, rel2)


# speedup vs baseline: 1.0086x; 1.0086x over previous
import jax
import jax.numpy as jnp
from jax import lax
from jax.experimental import pallas as pl
from jax.experimental.pallas import tpu as pltpu
from jax.experimental.pallas import tpu_sc as plsc

_NC = 2
_NS = 16
_NW = _NC * _NS
_L = 16

_BATCH = 4096
_BPW = _BATCH // _NW
_DIM = 128
_HALF = 64
_NUM_REL = 1000


def _cs_body(rel_ref, cs_ref):
    ph = rel_ref[...]
    cs_ref[:, :_HALF] = jnp.cos(ph)
    cs_ref[:, _HALF:] = jnp.sin(ph)


_cs_kernel = pl.pallas_call(
    _cs_body,
    out_shape=jax.ShapeDtypeStruct((_NUM_REL, _DIM), jnp.float32),
)


def _sc_body(h_hbm, r_hbm, t_hbm, ent_hbm, cs_hbm, out_hbm,
             h_idx, r_idx, t_idx, h_rows, t_rows, cs_rows, out_v,
             sem_h, sem_t, sem_p):
    wid = lax.axis_index("s") * _NC + lax.axis_index("c")
    base = wid * _BPW

    pltpu.sync_copy(h_hbm.at[pl.ds(base, _BPW)], h_idx)
    pltpu.sync_copy(r_hbm.at[pl.ds(base, _BPW)], r_idx)
    pltpu.sync_copy(t_hbm.at[pl.ds(base, _BPW)], t_idx)

    ch = pltpu.async_copy(ent_hbm.at[h_idx], h_rows, sem_h)
    ct = pltpu.async_copy(ent_hbm.at[t_idx], t_rows, sem_t)
    cp = pltpu.async_copy(cs_hbm.at[r_idx], cs_rows, sem_p)
    ch.wait()
    ct.wait()
    cp.wait()

    lanes = lax.iota(jnp.int32, _L)

    def g_step(g, carry):
        rows = g * _L + lanes

        def d_step(d, acc):
            dcol = (lanes + d) & (_HALF - 1)
            h_re = plsc.load_gather(h_rows, [rows, dcol])
            h_im = plsc.load_gather(h_rows, [rows, dcol + _HALF])
            t_re = plsc.load_gather(t_rows, [rows, dcol])
            t_im = plsc.load_gather(t_rows, [rows, dcol + _HALF])
            c = plsc.load_gather(cs_rows, [rows, dcol])
            s = plsc.load_gather(cs_rows, [rows, dcol + _HALF])
            d_re = h_re * c - h_im * s - t_re
            d_im = h_re * s + h_im * c - t_im
            return acc + d_re * d_re + d_im * d_im

        acc = lax.fori_loop(0, _HALF, d_step, jnp.zeros((_L,), jnp.float32),
                            unroll=4)

        bits = plsc.bitcast(acc, jnp.int32)
        y = plsc.bitcast(jnp.int32(0x5F3759DF) - (bits >> 1), jnp.float32)
        for _ in range(3):
            y = y * (1.5 - 0.5 * acc * y * y)
        root = jnp.where(acc > 0.0, acc * y, 0.0)
        out_v[pl.ds(pl.multiple_of(g * _L, _L), _L)] = -root
        return carry

    lax.fori_loop(0, _BPW // _L, g_step, 0)
    pltpu.sync_copy(out_v, out_hbm.at[pl.ds(base, _BPW)])


_sc_kernel = pl.kernel(
    _sc_body,
    out_type=jax.ShapeDtypeStruct((_BATCH,), jnp.float32),
    mesh=plsc.VectorSubcoreMesh(
        core_axis_name="c", subcore_axis_name="s",
        num_cores=_NC, num_subcores=_NS),
    scratch_types=[
        pltpu.VMEM((_BPW,), jnp.int32),
        pltpu.VMEM((_BPW,), jnp.int32),
        pltpu.VMEM((_BPW,), jnp.int32),
        pltpu.VMEM((_BPW, _DIM), jnp.float32),
        pltpu.VMEM((_BPW, _DIM), jnp.float32),
        pltpu.VMEM((_BPW, _DIM), jnp.float32),
        pltpu.VMEM((_BPW,), jnp.float32),
        pltpu.SemaphoreType.DMA,
        pltpu.SemaphoreType.DMA,
        pltpu.SemaphoreType.DMA,
    ],
    compiler_params=pltpu.CompilerParams(needs_layout_passes=False),
)


@jax.jit
def kernel(h, r, t, entity_embedding, relation_embedding):
    cs = _cs_kernel(relation_embedding)
    return _sc_kernel(h.astype(jnp.int32), r.astype(jnp.int32),
                      t.astype(jnp.int32), entity_embedding, cs)


# speedup vs baseline: 1.0748x; 1.0657x over previous
import jax
import jax.numpy as jnp
from jax import lax
from jax.experimental import pallas as pl
from jax.experimental.pallas import tpu as pltpu
from jax.experimental.pallas import tpu_sc as plsc

_NC = 2
_NS = 16
_NW = _NC * _NS
_L = 16

_BATCH = 4096
_BPW = _BATCH // _NW
_DIM = 128
_HALF = 64
_NUM_REL = 1000
_ACC = 4

_S1 = 0.9999788726879895
_S3 = -0.16649714106979646
_S5 = 0.007992247366759672
_C0 = 0.9999998110259923
_C2 = -0.49999394332144725
_C4 = 0.0416363038739887
_C6 = -0.001340053632153032


def _sc_body(h_hbm, r_hbm, t_hbm, ent_hbm, rel2_hbm, out_hbm,
             h_idx, r_idx, t_idx, r2_idx, h_rows, t_rows, p_rows, out_v,
             sem_h, sem_t, sem_p):
    wid = lax.axis_index("s") * _NC + lax.axis_index("c")
    base = wid * _BPW

    pltpu.sync_copy(h_hbm.at[pl.ds(base, _BPW)], h_idx)
    pltpu.sync_copy(r_hbm.at[pl.ds(base, _BPW)], r_idx)
    pltpu.sync_copy(t_hbm.at[pl.ds(base, _BPW)], t_idx)

    for k in range(_BPW // _L):
        r2_idx[pl.ds(k * _L, _L)] = r_idx[pl.ds(k * _L, _L)] >> 1

    ch = pltpu.async_copy(ent_hbm.at[h_idx], h_rows, sem_h)
    ct = pltpu.async_copy(ent_hbm.at[t_idx], t_rows, sem_t)
    cp = pltpu.async_copy(rel2_hbm.at[r2_idx], p_rows, sem_p)
    ch.wait()
    ct.wait()
    cp.wait()

    lanes = lax.iota(jnp.int32, _L)

    def g_step(g, carry):
        rows = g * _L + lanes
        rvals = r_idx[pl.ds(pl.multiple_of(g * _L, _L), _L)]
        p_off = (rvals & 1) << 6

        def d_step(i, accs):
            new = []
            for k in range(_ACC):
                acc = accs[k]
                dcol = (lanes + (i * _ACC + k)) & (_HALF - 1)
                h_re = plsc.load_gather(h_rows, [rows, dcol])
                h_im = plsc.load_gather(h_rows, [rows, dcol + _HALF])
                t_re = plsc.load_gather(t_rows, [rows, dcol])
                t_im = plsc.load_gather(t_rows, [rows, dcol + _HALF])
                ph = plsc.load_gather(p_rows, [rows, p_off + dcol])
                x2 = ph * ph
                s = ph * (_S1 + x2 * (_S3 + x2 * _S5))
                c = _C0 + x2 * (_C2 + x2 * (_C4 + x2 * _C6))
                d_re = h_re * c - h_im * s - t_re
                d_im = h_re * s + h_im * c - t_im
                new.append(acc + d_re * d_re + d_im * d_im)
            return tuple(new)

        zero = jnp.zeros((_L,), jnp.float32)
        accs = lax.fori_loop(0, _HALF // _ACC, d_step, (zero,) * _ACC)
        acc = (accs[0] + accs[1]) + (accs[2] + accs[3])

        bits = plsc.bitcast(acc, jnp.int32)
        y = plsc.bitcast(jnp.int32(0x5F3759DF) - (bits >> 1), jnp.float32)
        for _ in range(3):
            y = y * (1.5 - 0.5 * acc * y * y)
        root = jnp.where(acc > 0.0, acc * y, 0.0)
        out_v[pl.ds(pl.multiple_of(g * _L, _L), _L)] = -root
        return carry

    lax.fori_loop(0, _BPW // _L, g_step, 0)
    pltpu.sync_copy(out_v, out_hbm.at[pl.ds(base, _BPW)])


_sc_kernel = pl.kernel(
    _sc_body,
    out_type=jax.ShapeDtypeStruct((_BATCH,), jnp.float32),
    mesh=plsc.VectorSubcoreMesh(
        core_axis_name="c", subcore_axis_name="s",
        num_cores=_NC, num_subcores=_NS),
    scratch_types=[
        pltpu.VMEM((_BPW,), jnp.int32),
        pltpu.VMEM((_BPW,), jnp.int32),
        pltpu.VMEM((_BPW,), jnp.int32),
        pltpu.VMEM((_BPW,), jnp.int32),
        pltpu.VMEM((_BPW, _DIM), jnp.float32),
        pltpu.VMEM((_BPW, _DIM), jnp.float32),
        pltpu.VMEM((_BPW, _DIM), jnp.float32),
        pltpu.VMEM((_BPW,), jnp.float32),
        pltpu.SemaphoreType.DMA,
        pltpu.SemaphoreType.DMA,
        pltpu.SemaphoreType.DMA,
    ],
    compiler_params=pltpu.CompilerParams(needs_layout_passes=False),
)


@jax.jit
def kernel(h, r, t, entity_embedding, relation_embedding):
    rel2 = relation_embedding.reshape(_NUM_REL // 2, _DIM)
    return _sc_kernel(h.astype(jnp.int32), r.astype(jnp.int32),
                      t.astype(jnp.int32), entity_embedding, rel2)


# speedup vs baseline: 1.0941x; 1.0180x over previous
import jax
import jax.numpy as jnp
from jax import lax
from jax.experimental import pallas as pl
from jax.experimental.pallas import tpu as pltpu
from jax.experimental.pallas import tpu_sc as plsc

_NC = 2
_NS = 16
_NW = _NC * _NS
_L = 16

_BATCH = 4096
_BPW = _BATCH // _NW
_DIM = 128
_HALF = 64
_NUM_REL = 1000
_ACC = 4

_S1 = 0.9999788726879895
_S3 = -0.16649714106979646
_S5 = 0.007992247366759672
_C0 = 0.9999998110259923
_C2 = -0.49999394332144725
_C4 = 0.0416363038739887
_C6 = -0.001340053632153032


def _sc_body(h_hbm, r_hbm, t_hbm, ent_hbm, rel_hbm, out_hbm,
             h_idx, r_idx, t_idx, h_rows, t_rows, rel_v, out_v,
             sem_h, sem_t, sem_p):
    wid = lax.axis_index("s") * _NC + lax.axis_index("c")
    base = wid * _BPW

    pltpu.sync_copy(h_hbm.at[pl.ds(base, _BPW)], h_idx)
    pltpu.sync_copy(r_hbm.at[pl.ds(base, _BPW)], r_idx)
    pltpu.sync_copy(t_hbm.at[pl.ds(base, _BPW)], t_idx)

    ch = pltpu.async_copy(ent_hbm.at[h_idx], h_rows, sem_h)
    ct = pltpu.async_copy(ent_hbm.at[t_idx], t_rows, sem_t)
    cr = pltpu.async_copy(rel_hbm.at[r_idx], rel_v, sem_p)
    ch.wait()
    ct.wait()
    cr.wait()

    lanes = lax.iota(jnp.int32, _L)

    def g_step(g, carry):
        rows = g * _L + lanes
        rvals = r_idx[pl.ds(pl.multiple_of(g * _L, _L), _L)]

        def d_step(i, accs):
            new = []
            for k in range(_ACC):
                acc = accs[k]
                dcol = (lanes + (i * _ACC + k)) & (_HALF - 1)
                h_re = plsc.load_gather(h_rows, [rows, dcol])
                h_im = plsc.load_gather(h_rows, [rows, dcol + _HALF])
                t_re = plsc.load_gather(t_rows, [rows, dcol])
                t_im = plsc.load_gather(t_rows, [rows, dcol + _HALF])
                ph = plsc.load_gather(rel_v, [rows, dcol])
                x2 = ph * ph
                s = ph * (_S1 + x2 * (_S3 + x2 * _S5))
                c = _C0 + x2 * (_C2 + x2 * (_C4 + x2 * _C6))
                d_re = h_re * c - h_im * s - t_re
                d_im = h_re * s + h_im * c - t_im
                new.append(acc + d_re * d_re + d_im * d_im)
            return tuple(new)

        zero = jnp.zeros((_L,), jnp.float32)
        accs = lax.fori_loop(0, _HALF // _ACC, d_step, (zero,) * _ACC)
        acc = (accs[0] + accs[1]) + (accs[2] + accs[3])

        bits = plsc.bitcast(acc, jnp.int32)
        y = plsc.bitcast(jnp.int32(0x5F3759DF) - (bits >> 1), jnp.float32)
        for _ in range(3):
            y = y * (1.5 - 0.5 * acc * y * y)
        root = jnp.where(acc > 0.0, acc * y, 0.0)
        out_v[pl.ds(pl.multiple_of(g * _L, _L), _L)] = -root
        return carry

    lax.fori_loop(0, _BPW // _L, g_step, 0)
    pltpu.sync_copy(out_v, out_hbm.at[pl.ds(base, _BPW)])


_sc_kernel = pl.kernel(
    _sc_body,
    out_type=jax.ShapeDtypeStruct((_BATCH,), jnp.float32),
    mesh=plsc.VectorSubcoreMesh(
        core_axis_name="c", subcore_axis_name="s",
        num_cores=_NC, num_subcores=_NS),
    scratch_types=[
        pltpu.VMEM((_BPW,), jnp.int32),
        pltpu.VMEM((_BPW,), jnp.int32),
        pltpu.VMEM((_BPW,), jnp.int32),
        pltpu.VMEM((_BPW, _DIM), jnp.float32),
        pltpu.VMEM((_BPW, _DIM), jnp.float32),
        pltpu.VMEM((_BPW, _HALF), jnp.float32),
        pltpu.VMEM((_BPW,), jnp.float32),
        pltpu.SemaphoreType.DMA,
        pltpu.SemaphoreType.DMA,
        pltpu.SemaphoreType.DMA,
    ],
    compiler_params=pltpu.CompilerParams(needs_layout_passes=False, use_tc_tiling_on_sc=False),
)


@jax.jit
def kernel(h, r, t, entity_embedding, relation_embedding):
    return _sc_kernel(h.astype(jnp.int32), r.astype(jnp.int32),
                      t.astype(jnp.int32), entity_embedding,
                      relation_embedding)


# speedup vs baseline: 1.1031x; 1.0082x over previous
import jax
import jax.numpy as jnp
from jax import lax
from jax.experimental import pallas as pl
from jax.experimental.pallas import tpu as pltpu
from jax.experimental.pallas import tpu_sc as plsc

_NC = 2
_NS = 16
_NW = _NC * _NS
_L = 16

_BATCH = 4096
_BPW = _BATCH // _NW
_DIM = 128
_HALF = 64
_NUM_REL = 1000
_ACC = 4

_S1 = 0.9999788726879895
_S3 = -0.16649714106979646
_S5 = 0.007992247366759672
_C0 = 0.9999998110259923
_C2 = -0.49999394332144725
_C4 = 0.0416363038739887
_C6 = -0.001340053632153032


def _sc_body(h_hbm, r_hbm, t_hbm, ent_hbm, rel_hbm, out_hbm,
             h_idx, r_idx, t_idx, h_rows, t_rows, rel_v, c_rows, s_rows,
             out_v, sem_h, sem_t, sem_p):
    wid = lax.axis_index("s") * _NC + lax.axis_index("c")
    base = wid * _BPW

    pltpu.sync_copy(h_hbm.at[pl.ds(base, _BPW)], h_idx)
    pltpu.sync_copy(r_hbm.at[pl.ds(base, _BPW)], r_idx)
    pltpu.sync_copy(t_hbm.at[pl.ds(base, _BPW)], t_idx)

    cr = pltpu.async_copy(rel_hbm.at[r_idx], rel_v, sem_p)
    ch = pltpu.async_copy(ent_hbm.at[h_idx], h_rows, sem_h)
    ct = pltpu.async_copy(ent_hbm.at[t_idx], t_rows, sem_t)

    cr.wait()

    def cs_step(i, carry):
        for k in range(_HALF // _L):
            ph = rel_v[i, pl.ds(k * _L, _L)]
            x2 = ph * ph
            s_rows[i, pl.ds(k * _L, _L)] = ph * (_S1 + x2 * (_S3 + x2 * _S5))
            c_rows[i, pl.ds(k * _L, _L)] = _C0 + x2 * (_C2 + x2 * (_C4 + x2 * _C6))
        return carry

    lax.fori_loop(0, _BPW, cs_step, 0)

    ch.wait()
    ct.wait()

    lanes = lax.iota(jnp.int32, _L)

    def g_step(g, carry):
        rows = g * _L + lanes

        def d_step(i, accs):
            new = []
            for k in range(_ACC):
                acc = accs[k]
                dcol = (lanes + (i * _ACC + k)) & (_HALF - 1)
                h_re = plsc.load_gather(h_rows, [rows, dcol])
                h_im = plsc.load_gather(h_rows, [rows, dcol + _HALF])
                t_re = plsc.load_gather(t_rows, [rows, dcol])
                t_im = plsc.load_gather(t_rows, [rows, dcol + _HALF])
                c = plsc.load_gather(c_rows, [rows, dcol])
                s = plsc.load_gather(s_rows, [rows, dcol])
                d_re = h_re * c - h_im * s - t_re
                d_im = h_re * s + h_im * c - t_im
                new.append(acc + d_re * d_re + d_im * d_im)
            return tuple(new)

        zero = jnp.zeros((_L,), jnp.float32)
        accs = lax.fori_loop(0, _HALF // _ACC, d_step, (zero,) * _ACC)
        acc = (accs[0] + accs[1]) + (accs[2] + accs[3])

        bits = plsc.bitcast(acc, jnp.int32)
        y = plsc.bitcast(jnp.int32(0x5F3759DF) - (bits >> 1), jnp.float32)
        for _ in range(3):
            y = y * (1.5 - 0.5 * acc * y * y)
        root = jnp.where(acc > 0.0, acc * y, 0.0)
        out_v[pl.ds(pl.multiple_of(g * _L, _L), _L)] = -root
        return carry

    lax.fori_loop(0, _BPW // _L, g_step, 0)
    pltpu.sync_copy(out_v, out_hbm.at[pl.ds(base, _BPW)])


_sc_kernel = pl.kernel(
    _sc_body,
    out_type=jax.ShapeDtypeStruct((_BATCH,), jnp.float32),
    mesh=plsc.VectorSubcoreMesh(
        core_axis_name="c", subcore_axis_name="s",
        num_cores=_NC, num_subcores=_NS),
    scratch_types=[
        pltpu.VMEM((_BPW,), jnp.int32),
        pltpu.VMEM((_BPW,), jnp.int32),
        pltpu.VMEM((_BPW,), jnp.int32),
        pltpu.VMEM((_BPW, _DIM), jnp.float32),
        pltpu.VMEM((_BPW, _DIM), jnp.float32),
        pltpu.VMEM((_BPW, _HALF), jnp.float32),
        pltpu.VMEM((_BPW, _HALF), jnp.float32),
        pltpu.VMEM((_BPW, _HALF), jnp.float32),
        pltpu.VMEM((_BPW,), jnp.float32),
        pltpu.SemaphoreType.DMA,
        pltpu.SemaphoreType.DMA,
        pltpu.SemaphoreType.DMA,
    ],
    compiler_params=pltpu.CompilerParams(needs_layout_passes=False, use_tc_tiling_on_sc=False),
)


@jax.jit
def kernel(h, r, t, entity_embedding, relation_embedding):
    return _sc_kernel(h.astype(jnp.int32), r.astype(jnp.int32),
                      t.astype(jnp.int32), entity_embedding,
                      relation_embedding)
